# baseline (speedup 1.0000x reference)
import functools

import jax
import jax.numpy as jnp
from jax import lax
from jax.experimental import pallas as pl
from jax.experimental.pallas import tpu as pltpu


def _cdiv(a, b):
    return (a + b - 1) // b


def _focal_sum(o2d, t2d, gamma):
    diff = o2d - t2d
    sq = diff * diff
    row_loss = jnp.sum(sq, axis=-1, keepdims=True)
    w = 1.0 - jnp.exp(-row_loss)
    wg = w
    for _ in range(int(gamma) - 1):
        wg = wg * w
    focal = wg * row_loss
    return jnp.sum(focal, axis=0, keepdims=True)


def _kernel_3d(o_ref, t_ref, out_ref, *, gamma, bb):
    acc = _focal_sum(o_ref[0], t_ref[0], gamma)
    for j in range(1, bb):
        acc = acc + _focal_sum(o_ref[j], t_ref[j], gamma)
    out_ref[...] = acc.reshape(1, 1, 1)


def kernel(outputs, targets):
    gamma = 2
    B, S, D = outputs.shape
    n_items = B * S

    bb = 8
    while B % bb != 0:
        bb //= 2
    grid = B // bb

    kern = functools.partial(_kernel_3d, gamma=gamma, bb=bb)
    partials = pl.pallas_call(
        kern,
        out_shape=jax.ShapeDtypeStruct((grid, 1, 1), jnp.float32),
        grid_spec=pltpu.PrefetchScalarGridSpec(
            num_scalar_prefetch=0,
            grid=(grid,),
            in_specs=[
                pl.BlockSpec((bb, S, D), lambda i: (i, 0, 0)),
                pl.BlockSpec((bb, S, D), lambda i: (i, 0, 0)),
            ],
            out_specs=pl.BlockSpec((1, 1, 1), lambda i: (i, 0, 0)),
        ),
        compiler_params=pltpu.CompilerParams(
            dimension_semantics=("parallel",),
            vmem_limit_bytes=64 * 1024 * 1024,
        ),
    )(outputs, targets)
    return jnp.sum(partials) / float(n_items)


# speedup vs baseline: 4.2358x; 4.2358x over previous
import functools

import jax
import jax.numpy as jnp
from jax.experimental import pallas as pl
from jax.experimental.pallas import tpu as pltpu


def _focal_kernel(o_ref, t_ref, out_ref, *, gamma):
    diff = o_ref[...] - t_ref[...]
    sq = diff * diff
    row_loss = jnp.sum(sq, axis=1, keepdims=True)
    w = 1.0 - jnp.exp(-row_loss)
    wg = w
    for _ in range(int(gamma) - 1):
        wg = wg * w
    focal = wg * row_loss
    s = jnp.sum(focal, axis=2, keepdims=True)
    out_ref[...] = jnp.sum(s, axis=0, keepdims=True)


def kernel(outputs, targets):
    gamma = 2
    B, S, D = outputs.shape
    n_items = B * S

    o_t = outputs.transpose(0, 2, 1)
    t_t = targets.transpose(0, 2, 1)

    bb = 8
    while B % bb != 0:
        bb //= 2
    grid = B // bb

    kern = functools.partial(_focal_kernel, gamma=gamma)
    partials = pl.pallas_call(
        kern,
        out_shape=jax.ShapeDtypeStruct((grid, 1, 1), jnp.float32),
        grid_spec=pltpu.PrefetchScalarGridSpec(
            num_scalar_prefetch=0,
            grid=(grid,),
            in_specs=[
                pl.BlockSpec((bb, D, S), lambda i: (i, 0, 0)),
                pl.BlockSpec((bb, D, S), lambda i: (i, 0, 0)),
            ],
            out_specs=pl.BlockSpec((1, 1, 1), lambda i: (i, 0, 0)),
        ),
        compiler_params=pltpu.CompilerParams(
            dimension_semantics=("parallel",),
            vmem_limit_bytes=64 * 1024 * 1024,
        ),
    )(o_t, t_t)
    return jnp.sum(partials) / float(n_items)


# speedup vs baseline: 6.1209x; 1.4451x over previous
import functools

import jax
import jax.numpy as jnp
from jax.experimental import pallas as pl
from jax.experimental.pallas import tpu as pltpu


def _focal_kernel(o_ref, t_ref, out_ref, *, gamma):
    diff = o_ref[...] - t_ref[...]
    sq = diff * diff
    row_loss = jnp.sum(sq, axis=1, keepdims=True)
    w = 1.0 - jnp.exp(-row_loss)
    wg = w
    for _ in range(int(gamma) - 1):
        wg = wg * w
    focal = wg * row_loss
    s = jnp.sum(focal, axis=2, keepdims=True)
    out_ref[...] = jnp.sum(s, axis=0, keepdims=True)


def kernel(outputs, targets):
    gamma = 2
    B, S, D = outputs.shape
    n_items = B * S

    o_t = outputs.transpose(0, 2, 1)
    t_t = targets.transpose(0, 2, 1)

    bb = 32
    while B % bb != 0:
        bb //= 2
    grid = B // bb

    kern = functools.partial(_focal_kernel, gamma=gamma)
    partials = pl.pallas_call(
        kern,
        out_shape=jax.ShapeDtypeStruct((grid, 1, 1), jnp.float32),
        grid_spec=pltpu.PrefetchScalarGridSpec(
            num_scalar_prefetch=0,
            grid=(grid,),
            in_specs=[
                pl.BlockSpec((bb, D, S), lambda i: (i, 0, 0)),
                pl.BlockSpec((bb, D, S), lambda i: (i, 0, 0)),
            ],
            out_specs=pl.BlockSpec((1, 1, 1), lambda i: (i, 0, 0)),
        ),
        compiler_params=pltpu.CompilerParams(
            dimension_semantics=("parallel",),
            vmem_limit_bytes=64 * 1024 * 1024,
        ),
    )(o_t, t_t)
    return jnp.sum(partials) / float(n_items)
